# Initial kernel scaffold; baseline (speedup 1.0000x reference)
import jax
import jax.numpy as jnp
from jax.experimental import pallas as pl


def kernel(x, router_weight, w1, w2):
    raise NotImplementedError("write your pallas kernel here")


# speedup vs baseline: 3.3073x; 3.3073x over previous
import jax
import jax.numpy as jnp
from jax.experimental import pallas as pl
from jax.experimental.pallas import tpu as pltpu

_N = 2048
_H = 1024
_F = 4096
_E = 16
_K = 2
_BM = 256
_NBLK = _N * _K // _BM + _E
_NPAD = _NBLK * _BM
_GCH = 512
_TCH = 128
_ECH = 1024
_FSP = 2


def _meta_kernel(x_ref, rw_ref, ppc_ref, posc_ref, posr_ref, be_ref):
    xv = x_ref[...]
    rw = rw_ref[...]
    logits = jnp.dot(xv, rw, preferred_element_type=jnp.float32)
    mx = jnp.max(logits, axis=1, keepdims=True)
    ex = jnp.exp(logits - mx)
    probs = ex / jnp.sum(ex, axis=1, keepdims=True)
    col = jax.lax.broadcasted_iota(jnp.int32, (_N, _E), 1)
    m1 = jnp.max(probs, axis=1, keepdims=True)
    i1 = jnp.min(jnp.where(probs == m1, col, _E), axis=1, keepdims=True)
    pm = jnp.where(col == i1, -1.0, probs)
    m2 = jnp.max(pm, axis=1, keepdims=True)
    i2 = jnp.min(jnp.where(pm == m2, col, _E), axis=1, keepdims=True)
    s = m1 + m2
    p0 = m1 / s
    p1 = m2 / s
    a = (col == i1).astype(jnp.float32)
    b = (col == i2).astype(jnp.float32)
    t = a + b
    ri = jax.lax.broadcasted_iota(jnp.int32, (_N, _N), 0)
    ci = jax.lax.broadcasted_iota(jnp.int32, (_N, _N), 1)
    ltri = (ri > ci).astype(jnp.float32)
    pfx = jnp.dot(ltri, t, preferred_element_type=jnp.float32)
    counts = jnp.sum(t, axis=0, keepdims=True)
    padc = jnp.floor((counts + (_BM - 1)) / _BM) * _BM
    r16 = jax.lax.broadcasted_iota(jnp.int32, (_E, _E), 0)
    c16 = jax.lax.broadcasted_iota(jnp.int32, (_E, _E), 1)
    utri = (r16 < c16).astype(jnp.float32)
    off = jnp.dot(padc, utri, preferred_element_type=jnp.float32)
    pos1 = jnp.sum(a * (pfx + off), axis=1, keepdims=True)
    pos2 = jnp.sum(b * (pfx + off), axis=1, keepdims=True)
    ends = off + padc
    bk = (jax.lax.broadcasted_iota(jnp.int32, (_NBLK, _E), 0) * _BM).astype(jnp.float32)
    bev = jnp.sum((ends <= bk).astype(jnp.float32), axis=1, keepdims=True)
    bev = jnp.minimum(bev, _E - 1)
    colw = jax.lax.broadcasted_iota(jnp.int32, (_N, 128), 1)
    ppc_ref[...] = jnp.where(colw == 0, p0, jnp.where(colw == 1, p1, 0.0))
    posf = jnp.where(colw == 0, pos1, jnp.where(colw == 1, pos2, 0.0))
    posc_ref[...] = posf.astype(jnp.int32)
    pos8 = jnp.where(colw[:, :8] == 0, pos1, jnp.where(colw[:, :8] == 1, pos2, 0.0))
    posr_ref[...] = jnp.transpose(pos8).astype(jnp.int32)
    colb = jax.lax.broadcasted_iota(jnp.int32, (_NBLK, 128), 1)
    be_ref[...] = jnp.where(colb == 0, bev, 0.0).astype(jnp.int32)


def _gather_kernel(posr_ref, x_ref, xp_ref):
    base = pl.program_id(0) * _GCH
    p1r = posr_ref[0:1, :]
    p2r = posr_ref[1:2, :]
    rr = jax.lax.broadcasted_iota(jnp.int32, (_GCH, _N), 0) + base
    oh = ((rr == p1r) | (rr == p2r)).astype(jnp.float32)
    xp_ref[...] = jnp.dot(oh, x_ref[...], preferred_element_type=jnp.float32)


def _mlp_kernel(be_ref, xp_ref, w1_ref, w2_ref, eo_ref):
    f = pl.program_id(1)
    h = jnp.dot(xp_ref[...], w1_ref[0], preferred_element_type=jnp.float32)
    h = jax.nn.gelu(h)
    o = jnp.dot(h, w2_ref[0], preferred_element_type=jnp.float32)

    @pl.when(f == 0)
    def _():
        eo_ref[...] = o

    @pl.when(f != 0)
    def _():
        eo_ref[...] += o


def _combine_kernel(posc_ref, ppc_ref, eo_ref, out_ref):
    k = pl.program_id(1)
    pos1 = posc_ref[:, 0:1]
    pos2 = posc_ref[:, 1:2]
    p0 = ppc_ref[:, 0:1]
    p1 = ppc_ref[:, 1:2]
    rr = jax.lax.broadcasted_iota(jnp.int32, (_TCH, _ECH), 1) + k * _ECH
    w = jnp.where(pos1 == rr, p0, 0.0) + jnp.where(pos2 == rr, p1, 0.0)
    contrib = jnp.dot(w, eo_ref[...], preferred_element_type=jnp.float32)

    @pl.when(k == 0)
    def _():
        out_ref[...] = contrib

    @pl.when(k != 0)
    def _():
        out_ref[...] += contrib


def kernel(x, router_weight, w1, w2):
    f32 = jnp.float32
    i32 = jnp.int32
    ppc, posc, posr, be_arr = pl.pallas_call(
        _meta_kernel,
        out_shape=[
            jax.ShapeDtypeStruct((_N, 128), f32),
            jax.ShapeDtypeStruct((_N, 128), i32),
            jax.ShapeDtypeStruct((8, _N), i32),
            jax.ShapeDtypeStruct((_NBLK, 128), i32),
        ],
    )(x, router_weight)
    be_vec = be_arr[:, 0]

    xp = pl.pallas_call(
        _gather_kernel,
        grid=(_NPAD // _GCH,),
        in_specs=[
            pl.BlockSpec((8, _N), lambda m: (0, 0)),
            pl.BlockSpec((_N, _H), lambda m: (0, 0)),
        ],
        out_specs=pl.BlockSpec((_GCH, _H), lambda m: (m, 0)),
        out_shape=jax.ShapeDtypeStruct((_NPAD, _H), f32),
    )(posr, x)

    eo = pl.pallas_call(
        _mlp_kernel,
        grid_spec=pltpu.PrefetchScalarGridSpec(
            num_scalar_prefetch=1,
            grid=(_NBLK, _FSP),
            in_specs=[
                pl.BlockSpec((_BM, _H), lambda m, f, be: (m, 0)),
                pl.BlockSpec((1, _H, _F // _FSP), lambda m, f, be: (be[m], 0, f)),
                pl.BlockSpec((1, _F // _FSP, _H), lambda m, f, be: (be[m], f, 0)),
            ],
            out_specs=pl.BlockSpec((_BM, _H), lambda m, f, be: (m, 0)),
        ),
        out_shape=jax.ShapeDtypeStruct((_NPAD, _H), f32),
    )(be_vec, xp, w1, w2)

    out = pl.pallas_call(
        _combine_kernel,
        grid=(_N // _TCH, _NPAD // _ECH),
        in_specs=[
            pl.BlockSpec((_TCH, 128), lambda m, k: (m, 0)),
            pl.BlockSpec((_TCH, 128), lambda m, k: (m, 0)),
            pl.BlockSpec((_ECH, _H), lambda m, k: (k, 0)),
        ],
        out_specs=pl.BlockSpec((_TCH, _H), lambda m, k: (m, 0)),
        out_shape=jax.ShapeDtypeStruct((_N, _H), f32),
    )(posc, ppc, eo)
    return out
